# baseline (speedup 1.0000x reference)
import functools

import jax
import jax.numpy as jnp
from jax import lax
from jax.experimental import pallas as pl
from jax.experimental.pallas import tpu as pltpu
from jax.experimental.pallas import tpu_sc as plsc

N = 10000
D = 128
NPAD = 10240
ROWS_PER_SUB = NPAD // 16
NW = 32
EPB = 80
LAMDA = 0.5
R = 1000

_mesh = plsc.VectorSubcoreMesh(core_axis_name="c", subcore_axis_name="s")


def _sc_deg_body(nb, rows_hbm, zeros1_hbm, out_hbm, rowidx, ones_v, accd):
  c = lax.axis_index("c")
  s = lax.axis_index("s")
  wid = s * 2 + c
  pltpu.sync_copy(rows_hbm.at[wid], rowidx)
  pltpu.sync_copy(zeros1_hbm, accd.at[pl.ds(s * ROWS_PER_SUB, ROWS_PER_SUB)])
  for k in range(EPB // 16):
    ones_v[pl.ds(k * 16, 16)] = jnp.ones((16,), jnp.float32)
  plsc.subcore_barrier()

  def body(j, carry):
    pltpu.sync_copy(ones_v, accd.at[rowidx.at[j]], add=True)
    return carry

  lax.fori_loop(0, nb, body, 0)
  plsc.subcore_barrier()
  pltpu.sync_copy(accd.at[pl.ds(s * ROWS_PER_SUB, ROWS_PER_SUB)],
                  out_hbm.at[c, pl.ds(s * ROWS_PER_SUB, ROWS_PER_SUB)])


def _make_deg(nb):
  return pl.kernel(
      functools.partial(_sc_deg_body, nb),
      out_type=jax.ShapeDtypeStruct((2, NPAD), jnp.float32),
      mesh=_mesh,
      scratch_types=[
          pltpu.VMEM((nb, EPB), jnp.int32),
          pltpu.VMEM((EPB,), jnp.float32),
          pltpu.VMEM_SHARED((NPAD,), jnp.float32),
      ],
  )


def _sc_spmm_body(nb, epb, g_hbm, cols_hbm, rows_hbm, zeros2_hbm, out_hbm,
                  colidx, rowidx, rows_v0, rows_v1, acc, gsem0, gsem1):
  c = lax.axis_index("c")
  s = lax.axis_index("s")
  wid = s * 2 + c
  hb0 = nb // 2
  hb1 = nb - hb0
  pltpu.sync_copy(zeros2_hbm, acc.at[pl.ds(s * ROWS_PER_SUB, ROWS_PER_SUB)])
  plsc.subcore_barrier()

  def pair(jj, carry):
    ga = pltpu.async_copy(g_hbm.at[colidx.at[jj]], rows_v0, gsem0)
    gb = pltpu.async_copy(g_hbm.at[colidx.at[jj + 1]], rows_v1, gsem0)
    ga.wait()
    gb.wait()
    sa = pltpu.async_copy(rows_v0, acc.at[rowidx.at[jj]], gsem1, add=True)
    sb = pltpu.async_copy(rows_v1, acc.at[rowidx.at[jj + 1]], gsem1,
                          add=True)
    sa.wait()
    sb.wait()
    return carry

  def run_half(start, hb):
    pltpu.sync_copy(cols_hbm.at[wid, pl.ds(start, hb)], colidx)
    pltpu.sync_copy(rows_hbm.at[wid, pl.ds(start, hb)], rowidx)
    lax.fori_loop(0, hb // 2, lambda t, cc: pair(2 * t, cc), 0)

  run_half(0, hb0)
  run_half(hb0, hb1)
  plsc.subcore_barrier()
  pltpu.sync_copy(acc.at[pl.ds(s * ROWS_PER_SUB, ROWS_PER_SUB)],
                  out_hbm.at[c, pl.ds(s * ROWS_PER_SUB, ROWS_PER_SUB)])


def _make_spmm(nb, epb):
  return pl.kernel(
      functools.partial(_sc_spmm_body, nb, epb),
      out_type=jax.ShapeDtypeStruct((2, NPAD, D), jnp.float32),
      mesh=_mesh,
      scratch_types=[
          pltpu.VMEM((nb // 2, epb), jnp.int32),
          pltpu.VMEM((nb // 2, epb), jnp.int32),
          pltpu.VMEM((epb, D), jnp.float32),
          pltpu.VMEM((epb, D), jnp.float32),
          pltpu.VMEM_SHARED((NPAD, D), jnp.float32),
          pltpu.SemaphoreType.DMA,
          pltpu.SemaphoreType.DMA,
      ],
  )


def _init_body(degp_ref, x_ref, dinv_ref, g_ref):
  deg = degp_ref[0] + degp_ref[1]
  deg = jnp.where(deg == 0.0, 1.0, deg)
  dinv = lax.rsqrt(deg)
  dinv_ref[...] = dinv
  g_ref[...] = dinv * x_ref[...]


_init_call = pl.pallas_call(
    _init_body,
    grid=(N // R,),
    in_specs=[
        pl.BlockSpec((2, R, 1), lambda i: (0, i, 0)),
        pl.BlockSpec((R, D), lambda i: (i, 0)),
    ],
    out_specs=[
        pl.BlockSpec((R, 1), lambda i: (i, 0)),
        pl.BlockSpec((R, D), lambda i: (i, 0)),
    ],
    out_shape=[
        jax.ShapeDtypeStruct((N, 1), jnp.float32),
        jax.ShapeDtypeStruct((N, D), jnp.float32),
    ],
)


def _dense_body(theta, h_ref, aggp_ref, h0_ref, dinv_ref, W_ref, wqT_ref,
                bq1_ref, lng_ref, lnb_ref, h_out_ref, g_out_ref):
  h = h_ref[...]
  dinv = dinv_ref[...]
  s = jax.nn.sigmoid(
      jnp.sum(h * wqT_ref[...], axis=1, keepdims=True) + bq1_ref[0, 0])
  hi = dinv * (aggp_ref[0] + aggp_ref[1])
  support = (1.0 - s) * hi + s * h0_ref[...]
  out = theta * jnp.dot(support, W_ref[...],
                        preferred_element_type=jnp.float32) \
      + (1.0 - theta) * support
  r = jnp.maximum(out, 0.0)
  mu = jnp.mean(r, axis=1, keepdims=True)
  var = jnp.mean((r - mu) ** 2, axis=1, keepdims=True)
  hn = (r - mu) * lax.rsqrt(var + 1e-5) * lng_ref[...] + lnb_ref[...]
  h_out_ref[...] = hn
  g_out_ref[...] = dinv * hn


def _make_dense(theta):
  return pl.pallas_call(
      functools.partial(_dense_body, theta),
      grid=(N // R,),
      in_specs=[
          pl.BlockSpec((R, D), lambda i: (i, 0)),
          pl.BlockSpec((2, R, D), lambda i: (0, i, 0)),
          pl.BlockSpec((R, D), lambda i: (i, 0)),
          pl.BlockSpec((R, 1), lambda i: (i, 0)),
          pl.BlockSpec((D, D), lambda i: (0, 0)),
          pl.BlockSpec((1, D), lambda i: (0, 0)),
          pl.BlockSpec((1, 1), lambda i: (0, 0)),
          pl.BlockSpec((1, D), lambda i: (0, 0)),
          pl.BlockSpec((1, D), lambda i: (0, 0)),
      ],
      out_specs=[
          pl.BlockSpec((R, D), lambda i: (i, 0)),
          pl.BlockSpec((R, D), lambda i: (i, 0)),
      ],
      out_shape=[
          jax.ShapeDtypeStruct((N, D), jnp.float32),
          jax.ShapeDtypeStruct((N, D), jnp.float32),
      ],
  )


def _dense_final_body(theta, h_ref, aggp_ref, h0_ref, dinv_ref, W_ref,
                      wqT_ref, bq1_ref, lng_ref, lnb_ref, wc_ref, bc_ref,
                      o_ref):
  h = h_ref[...]
  dinv = dinv_ref[...]
  s = jax.nn.sigmoid(
      jnp.sum(h * wqT_ref[...], axis=1, keepdims=True) + bq1_ref[0, 0])
  hi = dinv * (aggp_ref[0] + aggp_ref[1])
  support = (1.0 - s) * hi + s * h0_ref[...]
  out = theta * jnp.dot(support, W_ref[...],
                        preferred_element_type=jnp.float32) \
      + (1.0 - theta) * support
  r = jnp.maximum(out, 0.0)
  mu = jnp.mean(r, axis=1, keepdims=True)
  var = jnp.mean((r - mu) ** 2, axis=1, keepdims=True)
  hn = (r - mu) * lax.rsqrt(var + 1e-5) * lng_ref[...] + lnb_ref[...]
  o_ref[...] = jnp.dot(hn, wc_ref[...],
                       preferred_element_type=jnp.float32) + bc_ref[...]


def _make_dense_final(theta):
  return pl.pallas_call(
      functools.partial(_dense_final_body, theta),
      grid=(N // R,),
      in_specs=[
          pl.BlockSpec((R, D), lambda i: (i, 0)),
          pl.BlockSpec((2, R, D), lambda i: (0, i, 0)),
          pl.BlockSpec((R, D), lambda i: (i, 0)),
          pl.BlockSpec((R, 1), lambda i: (i, 0)),
          pl.BlockSpec((D, D), lambda i: (0, 0)),
          pl.BlockSpec((1, D), lambda i: (0, 0)),
          pl.BlockSpec((1, 1), lambda i: (0, 0)),
          pl.BlockSpec((1, D), lambda i: (0, 0)),
          pl.BlockSpec((1, D), lambda i: (0, 0)),
          pl.BlockSpec((D, D), lambda i: (0, 0)),
          pl.BlockSpec((1, D), lambda i: (0, 0)),
      ],
      out_specs=pl.BlockSpec((R, D), lambda i: (i, 0)),
      out_shape=jax.ShapeDtypeStruct((N, D), jnp.float32),
  )


def kernel(x, adj, W0, W1, W2, W3, ln_gamma, ln_beta, wq, bq, wc, bc):
  E = adj.shape[1]
  nb_d = E // NW // EPB
  rows_d = adj[0].reshape(NW, nb_d, EPB)
  epw = -(-E // (NW * 4 * EPB)) * 4 * EPB
  nb = epw // EPB
  pad = NW * epw - E
  pad_rows = jnp.arange(pad, dtype=adj.dtype) % N
  rows = jnp.concatenate([adj[0], pad_rows]).reshape(NW, nb, EPB)
  cols = jnp.pad(adj[1], (0, pad), constant_values=N).reshape(NW, nb, EPB)
  zrow = jnp.zeros((8, D), jnp.float32)
  zeros1 = jnp.zeros((ROWS_PER_SUB,), jnp.float32)
  zeros2 = jnp.zeros((ROWS_PER_SUB, D), jnp.float32)

  degp = _make_deg(nb_d)(rows_d, zeros1)
  degp3 = degp.reshape(2, NPAD, 1)
  dinv, g = _init_call(degp3, x)

  wqT = wq.reshape(1, D)
  bq1 = (bq - 1.0).reshape(1, 1)
  lng = ln_gamma.reshape(1, D)
  lnb = ln_beta.reshape(1, D)
  wc_pad = jnp.pad(wc, ((0, 0), (0, D - wc.shape[1])))
  bc_pad = jnp.pad(bc, (0, D - bc.shape[0])).reshape(1, D)

  spmm = _make_spmm(nb, EPB)
  h = x
  for i, W in enumerate([W0, W1, W2]):
    g1 = jnp.concatenate([g, zrow])
    aggp = spmm(g1, cols, rows, zeros2)
    h, g = _make_dense(LAMDA / (i + 1))(
        h, aggp, x, dinv, W, wqT, bq1, lng, lnb)

  g1 = jnp.concatenate([g, zrow])
  aggp = spmm(g1, cols, rows, zeros2)
  logits_pad = _make_dense_final(LAMDA / 4)(
      h, aggp, x, dinv, W3, wqT, bq1, lng, lnb, wc_pad, bc_pad)
  return logits_pad[:, :wc.shape[1]]


# speedup vs baseline: 2.1352x; 2.1352x over previous
import functools

import jax
import jax.numpy as jnp
from jax import lax
from jax.experimental import pallas as pl
from jax.experimental.pallas import tpu as pltpu
from jax.experimental.pallas import tpu_sc as plsc

N = 10000
D = 128
NPAD = 10240
ROWS_PER_SUB = NPAD // 16
NW = 32
EPB = 80
LAMDA = 0.5
R = 1000

_mesh = plsc.VectorSubcoreMesh(core_axis_name="c", subcore_axis_name="s")


def _sc_deg_body(nb, rows_hbm, zeros1_hbm, out_hbm, rowidx, ones_v, accd):
  c = lax.axis_index("c")
  s = lax.axis_index("s")
  wid = s * 2 + c
  pltpu.sync_copy(rows_hbm.at[wid], rowidx)
  pltpu.sync_copy(zeros1_hbm, accd.at[pl.ds(s * ROWS_PER_SUB, ROWS_PER_SUB)])
  for k in range(EPB // 16):
    ones_v[pl.ds(k * 16, 16)] = jnp.ones((16,), jnp.float32)
  plsc.subcore_barrier()

  def body(j, carry):
    pltpu.sync_copy(ones_v, accd.at[rowidx.at[j]], add=True)
    return carry

  lax.fori_loop(0, nb, body, 0)
  plsc.subcore_barrier()
  pltpu.sync_copy(accd.at[pl.ds(s * ROWS_PER_SUB, ROWS_PER_SUB)],
                  out_hbm.at[c, pl.ds(s * ROWS_PER_SUB, ROWS_PER_SUB)])


def _make_deg(nb):
  return pl.kernel(
      functools.partial(_sc_deg_body, nb),
      out_type=jax.ShapeDtypeStruct((2, NPAD), jnp.float32),
      mesh=_mesh,
      scratch_types=[
          pltpu.VMEM((nb, EPB), jnp.int32),
          pltpu.VMEM((EPB,), jnp.float32),
          pltpu.VMEM_SHARED((NPAD,), jnp.float32),
      ],
  )


def _sc_spmm_body(nb, epb, g_hbm, cols_hbm, rows_hbm, zeros2_hbm, out_hbm,
                  colidx, rowidx, rows_v, acc):
  c = lax.axis_index("c")
  s = lax.axis_index("s")
  wid = s * 2 + c
  pltpu.sync_copy(cols_hbm.at[wid], colidx)
  pltpu.sync_copy(rows_hbm.at[wid], rowidx)
  pltpu.sync_copy(zeros2_hbm, acc.at[pl.ds(s * ROWS_PER_SUB, ROWS_PER_SUB)])
  plsc.subcore_barrier()

  def body(j, carry):
    pltpu.sync_copy(g_hbm.at[colidx.at[j]], rows_v)
    pltpu.sync_copy(rows_v, acc.at[rowidx.at[j]], add=True)
    return carry

  lax.fori_loop(0, nb, body, 0)
  plsc.subcore_barrier()
  pltpu.sync_copy(acc.at[pl.ds(s * ROWS_PER_SUB, ROWS_PER_SUB)],
                  out_hbm.at[c, pl.ds(s * ROWS_PER_SUB, ROWS_PER_SUB)])


def _make_spmm(nb, epb):
  return pl.kernel(
      functools.partial(_sc_spmm_body, nb, epb),
      out_type=jax.ShapeDtypeStruct((2, NPAD, D), jnp.float32),
      mesh=_mesh,
      scratch_types=[
          pltpu.VMEM((nb, epb), jnp.int32),
          pltpu.VMEM((nb, epb), jnp.int32),
          pltpu.VMEM((epb, D), jnp.float32),
          pltpu.VMEM_SHARED((NPAD, D), jnp.float32),
      ],
  )


def _init_body(degp_ref, x_ref, dinv_ref, g_ref):
  deg = degp_ref[0] + degp_ref[1]
  deg = jnp.where(deg == 0.0, 1.0, deg)
  dinv = lax.rsqrt(deg)
  dinv_ref[...] = dinv
  g_ref[...] = dinv * x_ref[...]


_init_call = pl.pallas_call(
    _init_body,
    grid=(N // R,),
    in_specs=[
        pl.BlockSpec((2, R, 1), lambda i: (0, i, 0)),
        pl.BlockSpec((R, D), lambda i: (i, 0)),
    ],
    out_specs=[
        pl.BlockSpec((R, 1), lambda i: (i, 0)),
        pl.BlockSpec((R, D), lambda i: (i, 0)),
    ],
    out_shape=[
        jax.ShapeDtypeStruct((N, 1), jnp.float32),
        jax.ShapeDtypeStruct((N, D), jnp.float32),
    ],
)


def _dense_body(theta, h_ref, aggp_ref, h0_ref, dinv_ref, W_ref, wqT_ref,
                bq1_ref, lng_ref, lnb_ref, h_out_ref, g_out_ref):
  h = h_ref[...]
  dinv = dinv_ref[...]
  s = jax.nn.sigmoid(
      jnp.sum(h * wqT_ref[...], axis=1, keepdims=True) + bq1_ref[0, 0])
  hi = dinv * (aggp_ref[0] + aggp_ref[1])
  support = (1.0 - s) * hi + s * h0_ref[...]
  out = theta * jnp.dot(support, W_ref[...],
                        preferred_element_type=jnp.float32) \
      + (1.0 - theta) * support
  r = jnp.maximum(out, 0.0)
  mu = jnp.mean(r, axis=1, keepdims=True)
  var = jnp.mean((r - mu) ** 2, axis=1, keepdims=True)
  hn = (r - mu) * lax.rsqrt(var + 1e-5) * lng_ref[...] + lnb_ref[...]
  h_out_ref[...] = hn
  g_out_ref[...] = dinv * hn


def _make_dense(theta):
  return pl.pallas_call(
      functools.partial(_dense_body, theta),
      grid=(N // R,),
      in_specs=[
          pl.BlockSpec((R, D), lambda i: (i, 0)),
          pl.BlockSpec((2, R, D), lambda i: (0, i, 0)),
          pl.BlockSpec((R, D), lambda i: (i, 0)),
          pl.BlockSpec((R, 1), lambda i: (i, 0)),
          pl.BlockSpec((D, D), lambda i: (0, 0)),
          pl.BlockSpec((1, D), lambda i: (0, 0)),
          pl.BlockSpec((1, 1), lambda i: (0, 0)),
          pl.BlockSpec((1, D), lambda i: (0, 0)),
          pl.BlockSpec((1, D), lambda i: (0, 0)),
      ],
      out_specs=[
          pl.BlockSpec((R, D), lambda i: (i, 0)),
          pl.BlockSpec((R, D), lambda i: (i, 0)),
      ],
      out_shape=[
          jax.ShapeDtypeStruct((N, D), jnp.float32),
          jax.ShapeDtypeStruct((N, D), jnp.float32),
      ],
  )


def _dense_final_body(theta, h_ref, aggp_ref, h0_ref, dinv_ref, W_ref,
                      wqT_ref, bq1_ref, lng_ref, lnb_ref, wc_ref, bc_ref,
                      o_ref):
  h = h_ref[...]
  dinv = dinv_ref[...]
  s = jax.nn.sigmoid(
      jnp.sum(h * wqT_ref[...], axis=1, keepdims=True) + bq1_ref[0, 0])
  hi = dinv * (aggp_ref[0] + aggp_ref[1])
  support = (1.0 - s) * hi + s * h0_ref[...]
  out = theta * jnp.dot(support, W_ref[...],
                        preferred_element_type=jnp.float32) \
      + (1.0 - theta) * support
  r = jnp.maximum(out, 0.0)
  mu = jnp.mean(r, axis=1, keepdims=True)
  var = jnp.mean((r - mu) ** 2, axis=1, keepdims=True)
  hn = (r - mu) * lax.rsqrt(var + 1e-5) * lng_ref[...] + lnb_ref[...]
  o_ref[...] = jnp.dot(hn, wc_ref[...],
                       preferred_element_type=jnp.float32) + bc_ref[...]


def _make_dense_final(theta):
  return pl.pallas_call(
      functools.partial(_dense_final_body, theta),
      grid=(N // R,),
      in_specs=[
          pl.BlockSpec((R, D), lambda i: (i, 0)),
          pl.BlockSpec((2, R, D), lambda i: (0, i, 0)),
          pl.BlockSpec((R, D), lambda i: (i, 0)),
          pl.BlockSpec((R, 1), lambda i: (i, 0)),
          pl.BlockSpec((D, D), lambda i: (0, 0)),
          pl.BlockSpec((1, D), lambda i: (0, 0)),
          pl.BlockSpec((1, 1), lambda i: (0, 0)),
          pl.BlockSpec((1, D), lambda i: (0, 0)),
          pl.BlockSpec((1, D), lambda i: (0, 0)),
          pl.BlockSpec((D, D), lambda i: (0, 0)),
          pl.BlockSpec((1, D), lambda i: (0, 0)),
      ],
      out_specs=pl.BlockSpec((R, D), lambda i: (i, 0)),
      out_shape=jax.ShapeDtypeStruct((N, D), jnp.float32),
  )


def kernel(x, adj, W0, W1, W2, W3, ln_gamma, ln_beta, wq, bq, wc, bc):
  E = adj.shape[1]
  epw = E // NW
  nb = epw // EPB
  rows = adj[0].reshape(NW, nb, EPB)
  cols = adj[1].reshape(NW, nb, EPB)
  zeros1 = jnp.zeros((ROWS_PER_SUB,), jnp.float32)
  zeros2 = jnp.zeros((ROWS_PER_SUB, D), jnp.float32)

  degp = _make_deg(nb)(rows, zeros1)
  degp3 = degp.reshape(2, NPAD, 1)
  dinv, g = _init_call(degp3, x)

  wqT = wq.reshape(1, D)
  bq1 = (bq - 1.0).reshape(1, 1)
  lng = ln_gamma.reshape(1, D)
  lnb = ln_beta.reshape(1, D)
  wc_pad = jnp.pad(wc, ((0, 0), (0, D - wc.shape[1])))
  bc_pad = jnp.pad(bc, (0, D - bc.shape[0])).reshape(1, D)

  spmm = _make_spmm(nb, EPB)
  h = x
  for i, W in enumerate([W0, W1, W2]):
    aggp = spmm(g, cols, rows, zeros2)
    h, g = _make_dense(LAMDA / (i + 1))(
        h, aggp, x, dinv, W, wqT, bq1, lng, lnb)

  aggp = spmm(g, cols, rows, zeros2)
  logits_pad = _make_dense_final(LAMDA / 4)(
      h, aggp, x, dinv, W3, wqT, bq1, lng, lnb, wc_pad, bc_pad)
  return logits_pad[:, :wc.shape[1]]
